# baseline (speedup 1.0000x reference)
import functools

import jax
import jax.numpy as jnp
from jax import lax
from jax.experimental import pallas as pl
from jax.experimental.pallas import tpu as pltpu
from jax.experimental.pallas import tpu_sc as plsc

B = 16384
EMBED = 64
D_DENSE = 128
D_HID = 256
D_OUT = 64

_NC = 2
_NS = 16
_NW = _NC * _NS
_BPW = B // _NW
_CHUNK = 128
_NCHUNK = _BPW // _CHUNK


def _sc_gather_body(table_hbm, idx_hbm, out_hbm, idx_v, rows_v, sem):
    wid = lax.axis_index("s") * _NC + lax.axis_index("c")
    base = wid * _BPW
    pltpu.sync_copy(idx_hbm.at[wid], idx_v)
    copies = []
    for j in range(_NCHUNK):
        copies.append(
            pltpu.async_copy(
                table_hbm.at[idx_v.at[j]],
                rows_v.at[pl.ds(j * _CHUNK, _CHUNK)],
                sem,
            )
        )
    for c in copies:
        c.wait()
    pltpu.sync_copy(rows_v, out_hbm.at[pl.ds(base, _BPW)])


def _sc_gather(em_table, indices):
    mesh = plsc.VectorSubcoreMesh(core_axis_name="c", subcore_axis_name="s")
    idx2d = indices.reshape(_NW, _NCHUNK, _CHUNK).astype(jnp.int32)
    k = pl.kernel(
        _sc_gather_body,
        mesh=mesh,
        out_type=jax.ShapeDtypeStruct((B, EMBED), jnp.float32),
        scratch_types=[
            pltpu.VMEM((_NCHUNK, _CHUNK), jnp.int32),
            pltpu.VMEM((_BPW, EMBED), jnp.float32),
            pltpu.SemaphoreType.DMA,
        ],
        compiler_params=pltpu.CompilerParams(use_tc_tiling_on_sc=False),
    )
    return k(em_table, idx2d)


_BR = 2048


def _mlp_body(dense_ref, sp_ref, w1_ref, b1_ref, wn_ref, bn_ref, w2_ref,
              b2_ref, out_ref):
    t = jnp.dot(dense_ref[:], w1_ref[:], preferred_element_type=jnp.float32)
    t = t + b1_ref[:]
    h = jnp.dot(t, wn_ref[:D_DENSE, :], preferred_element_type=jnp.float32)
    h = h + jnp.dot(sp_ref[:], wn_ref[D_DENSE:, :],
                    preferred_element_type=jnp.float32)
    h = jnp.maximum(h + bn_ref[:], 0.0)
    o = jnp.dot(h, w2_ref[:], preferred_element_type=jnp.float32)
    out_ref[:] = o + b2_ref[:]


def _mlp(dense_features, sparse_rows, W1, b1, Wn, bn, W2, b2):
    grid = (B // _BR,)
    return pl.pallas_call(
        _mlp_body,
        grid=grid,
        in_specs=[
            pl.BlockSpec((_BR, D_DENSE), lambda i: (i, 0)),
            pl.BlockSpec((_BR, EMBED), lambda i: (i, 0)),
            pl.BlockSpec((D_DENSE, D_DENSE), lambda i: (0, 0)),
            pl.BlockSpec((1, D_DENSE), lambda i: (0, 0)),
            pl.BlockSpec((D_DENSE + EMBED, D_HID), lambda i: (0, 0)),
            pl.BlockSpec((1, D_HID), lambda i: (0, 0)),
            pl.BlockSpec((D_HID, D_OUT), lambda i: (0, 0)),
            pl.BlockSpec((1, D_OUT), lambda i: (0, 0)),
        ],
        out_specs=pl.BlockSpec((_BR, D_OUT), lambda i: (i, 0)),
        out_shape=jax.ShapeDtypeStruct((B, D_OUT), jnp.float32),
    )(dense_features, sparse_rows, W1, b1.reshape(1, -1), Wn,
      bn.reshape(1, -1), W2, b2.reshape(1, -1))


def kernel(dense_features, sparse_features, labels, em_table, W1, b1, Wn, bn,
           W2, b2):
    sparse_rows = _sc_gather(em_table, sparse_features)
    return _mlp(dense_features, sparse_rows, W1, b1, Wn, bn, W2, b2)


# speedup vs baseline: 2.0027x; 2.0027x over previous
import functools

import jax
import jax.numpy as jnp
from jax import lax
from jax.experimental import pallas as pl
from jax.experimental.pallas import tpu as pltpu
from jax.experimental.pallas import tpu_sc as plsc

B = 16384
VOCAB = 1000000
EMBED = 64
D_DENSE = 128
D_HID = 256
D_OUT = 64

_CB = 2048
_SPLIT = 244 * _CB
_NPAIR = VOCAB - _SPLIT
_TGRID = (_NPAIR + _CB - 1) // _CB


def _transpose_body(lo_ref, hi_ref, eye_ref, out_ref):
    stack = jnp.concatenate([lo_ref[:], hi_ref[:]], axis=0)
    dn = (((0,), (0,)), ((), ()))
    out_ref[:] = lax.dot_general(
        stack, eye_ref[:], dn, preferred_element_type=jnp.float32)


def _build_pairs(table_t, eye):
    return pl.pallas_call(
        _transpose_body,
        grid=(_TGRID,),
        in_specs=[
            pl.BlockSpec((EMBED, _CB), lambda g: (0, g)),
            pl.BlockSpec((EMBED, _CB), lambda g: (0, 244 + g)),
            pl.BlockSpec((2 * EMBED, 2 * EMBED), lambda g: (0, 0)),
        ],
        out_specs=pl.BlockSpec((_CB, 2 * EMBED), lambda g: (g, 0)),
        out_shape=jax.ShapeDtypeStruct((_NPAIR, 2 * EMBED), jnp.float32),
    )(table_t, table_t, eye)


_NC = 2
_NS = 16
_NW = _NC * _NS
_BPW = B // _NW
_CHUNK = 128
_NCHUNK = _BPW // _CHUNK


def _sc_gather_body(table_hbm, idx_hbm, out_hbm, idx_v, rows_v, sem):
    wid = lax.axis_index("s") * _NC + lax.axis_index("c")
    base = wid * _BPW
    pltpu.sync_copy(idx_hbm.at[wid], idx_v)
    copies = []
    for j in range(_NCHUNK):
        copies.append(
            pltpu.async_copy(
                table_hbm.at[idx_v.at[j]],
                rows_v.at[pl.ds(j * _CHUNK, _CHUNK)],
                sem,
            )
        )
    for c in copies:
        c.wait()
    pltpu.sync_copy(rows_v, out_hbm.at[pl.ds(base, _BPW)])


def _sc_gather(table_pairs, idx2):
    mesh = plsc.VectorSubcoreMesh(core_axis_name="c", subcore_axis_name="s")
    k = pl.kernel(
        _sc_gather_body,
        mesh=mesh,
        out_type=jax.ShapeDtypeStruct((B, 2 * EMBED), jnp.float32),
        scratch_types=[
            pltpu.VMEM((_NCHUNK, _CHUNK), jnp.int32),
            pltpu.VMEM((_BPW, 2 * EMBED), jnp.float32),
            pltpu.SemaphoreType.DMA,
        ],
        compiler_params=pltpu.CompilerParams(use_tc_tiling_on_sc=False),
    )
    return k(table_pairs, idx2)


_BR = 2048


def _mlp_body(dense_ref, p_ref, par_ref, w1_ref, b1_ref, wna_ref, wnb2_ref,
              bn_ref, w2_ref, b2_ref, out_ref):
    t = jnp.dot(dense_ref[:], w1_ref[:], preferred_element_type=jnp.float32)
    t = t + b1_ref[:]
    h = jnp.dot(t, wna_ref[:], preferred_element_type=jnp.float32)
    q = jnp.dot(p_ref[:], wnb2_ref[:], preferred_element_type=jnp.float32)
    sp = jnp.where(par_ref[:] > 0.5, q[:, D_HID:], q[:, :D_HID])
    h = jnp.maximum(h + sp + bn_ref[:], 0.0)
    o = jnp.dot(h, w2_ref[:], preferred_element_type=jnp.float32)
    out_ref[:] = o + b2_ref[:]


def _mlp(dense_features, pairs, par, W1, b1, WnA, WnB2, bn, W2, b2):
    grid = (B // _BR,)
    return pl.pallas_call(
        _mlp_body,
        grid=grid,
        in_specs=[
            pl.BlockSpec((_BR, D_DENSE), lambda i: (i, 0)),
            pl.BlockSpec((_BR, 2 * EMBED), lambda i: (i, 0)),
            pl.BlockSpec((_BR, 1), lambda i: (i, 0)),
            pl.BlockSpec((D_DENSE, D_DENSE), lambda i: (0, 0)),
            pl.BlockSpec((1, D_DENSE), lambda i: (0, 0)),
            pl.BlockSpec((D_DENSE, D_HID), lambda i: (0, 0)),
            pl.BlockSpec((2 * EMBED, 2 * D_HID), lambda i: (0, 0)),
            pl.BlockSpec((1, D_HID), lambda i: (0, 0)),
            pl.BlockSpec((D_HID, D_OUT), lambda i: (0, 0)),
            pl.BlockSpec((1, D_OUT), lambda i: (0, 0)),
        ],
        out_specs=pl.BlockSpec((_BR, D_OUT), lambda i: (i, 0)),
        out_shape=jax.ShapeDtypeStruct((B, D_OUT), jnp.float32),
    )(dense_features, pairs, par, W1, b1.reshape(1, -1), WnA, WnB2,
      bn.reshape(1, -1), W2, b2.reshape(1, -1))


def kernel(dense_features, sparse_features, labels, em_table, W1, b1, Wn, bn,
           W2, b2):
    idx = sparse_features.astype(jnp.int32)
    pairs_table = _build_pairs(em_table.T,
                               jnp.eye(2 * EMBED, dtype=jnp.float32))
    in_hi = idx >= _SPLIT
    row = jnp.where(in_hi, idx - _SPLIT, idx)
    idx2 = row.reshape(_NW, _NCHUNK, _CHUNK)
    pairs = _sc_gather(pairs_table, idx2)
    par = in_hi.astype(jnp.float32).reshape(B, 1)
    WnA = Wn[:D_DENSE]
    WnB = Wn[D_DENSE:]
    WnB2 = jnp.zeros((2 * EMBED, 2 * D_HID), jnp.float32)
    WnB2 = WnB2.at[:EMBED, :D_HID].set(WnB).at[EMBED:, D_HID:].set(WnB)
    return _mlp(dense_features, pairs, par, W1, b1, WnA, WnB2, bn, W2, b2)
